# baseline (speedup 1.0000x reference)
import functools

import jax
import jax.numpy as jnp
from jax import lax
from jax.experimental import pallas as pl
from jax.experimental.pallas import tpu as pltpu
from jax.experimental.pallas import tpu_sc as plsc


def kernel(inputs, table):
    B, S = inputs.shape
    N, D = table.shape
    info = plsc.get_sparse_core_info()
    NC, NS = info.num_cores, info.num_subcores
    NW = NC * NS
    RW = S // NW

    mesh = plsc.VectorSubcoreMesh(core_axis_name="c", subcore_axis_name="s")

    @functools.partial(
        pl.kernel,
        mesh=mesh,
        out_type=jax.ShapeDtypeStruct((B, S, D), table.dtype),
        scratch_types=[pltpu.SemaphoreType.DMA],
    )
    def run(table_hbm, out_hbm, sem):
        wid = lax.axis_index("s") * NC + lax.axis_index("c")
        base = wid * RW
        hs = [
            pltpu.async_copy(
                table_hbm.at[pl.ds(base, RW)],
                out_hbm.at[b, pl.ds(base, RW)],
                sem,
            )
            for b in range(B)
        ]
        for h in hs:
            h.wait()

    return run(table)


# speedup vs baseline: 23.6133x; 23.6133x over previous
import functools

import jax
import jax.numpy as jnp
from jax import lax
from jax.experimental import pallas as pl
from jax.experimental.pallas import tpu as pltpu
from jax.experimental.pallas import tpu_sc as plsc


def _tc_body(t_ref, o_ref):
    o_ref[...] = jnp.broadcast_to(t_ref[...][None], o_ref.shape)


def kernel(inputs, table):
    B, S = inputs.shape
    N, D = table.shape
    BH = B // 2
    BLK = 512
    tc_half = pl.pallas_call(
        _tc_body,
        grid=(S // BLK,),
        in_specs=[pl.BlockSpec((BLK, D), lambda j: (j, 0))],
        out_specs=pl.BlockSpec((BH, BLK, D), lambda j: (0, j, 0)),
        out_shape=jax.ShapeDtypeStruct((BH, S, D), table.dtype),
    )(table)

    info = plsc.get_sparse_core_info()
    NC, NS = info.num_cores, info.num_subcores
    NW = NC * NS
    RW = S // NW
    CHUNK = 128
    NCH = RW // CHUNK
    mesh = plsc.VectorSubcoreMesh(core_axis_name="c", subcore_axis_name="s")

    @functools.partial(
        pl.kernel,
        mesh=mesh,
        out_type=jax.ShapeDtypeStruct((BH, S, D), table.dtype),
        scratch_types=[pltpu.VMEM((CHUNK, D), jnp.float32)],
    )
    def sc_half(table_hbm, out_hbm, buf):
        wid = lax.axis_index("s") * NC + lax.axis_index("c")
        base = wid * RW
        for k in range(NCH):
            row0 = base + k * CHUNK
            pltpu.sync_copy(table_hbm.at[pl.ds(row0, CHUNK)], buf)
            for b in range(BH):
                pltpu.sync_copy(buf, out_hbm.at[b, pl.ds(row0, CHUNK)])

    return jnp.concatenate([tc_half, sc_half(table)], axis=0)


# speedup vs baseline: 51.9246x; 2.1990x over previous
import functools

import jax
import jax.numpy as jnp
from jax import lax
from jax.experimental import pallas as pl
from jax.experimental.pallas import tpu as pltpu
from jax.experimental.pallas import tpu_sc as plsc


def kernel(inputs, table):
    B, S = inputs.shape
    N, D = table.shape
    info = plsc.get_sparse_core_info()
    NC, NS = info.num_cores, info.num_subcores
    NW = NC * NS
    RW = S // NW
    CHUNK = 128
    NCH = RW // CHUNK

    mesh = plsc.VectorSubcoreMesh(core_axis_name="c", subcore_axis_name="s")

    @functools.partial(
        pl.kernel,
        mesh=mesh,
        out_type=jax.ShapeDtypeStruct((B, S, D), table.dtype),
        scratch_types=[pltpu.VMEM((CHUNK, D), jnp.float32)],
    )
    def run(table_hbm, out_hbm, buf):
        wid = lax.axis_index("s") * NC + lax.axis_index("c")
        base = wid * RW
        for k in range(NCH):
            row0 = base + k * CHUNK
            pltpu.sync_copy(table_hbm.at[pl.ds(row0, CHUNK)], buf)
            for b in range(B):
                pltpu.sync_copy(buf, out_hbm.at[b, pl.ds(row0, CHUNK)])

    return run(table)
